# Initial kernel scaffold; baseline (speedup 1.0000x reference)
import math

import jax
import jax.numpy as jnp
import numpy as np
from jax import lax
from jax.experimental import pallas as pl
from jax.experimental.pallas import tpu as pltpu

H, W = 512, 1024
EPS = 1e-05
B, F = 64, 50
N = H * W


def _row_tables():
    thetas = np.linspace(0.5, H - 0.5, num=H) * math.pi / H
    weight = 1.0 / np.sin(thetas)
    residual = weight % 2
    mask = residual >= 1
    residual[mask] -= 1
    residual[~mask] += 1
    n_ones = (weight - residual).astype(np.int32)
    edge_values = ((weight - n_ones) / 2).astype(np.float32)
    lengths = n_ones + 2
    return lengths.astype(np.int32), edge_values


_LEN_NP, _EV_NP = _row_tables()


def _nss_kernel(rows_ref, lefts_ref, widths_ref, evs_ref, a_ref, out_ref, fm_ref):
    a = a_ref[0]
    s1 = jnp.sum(a)
    s2 = jnp.sum(a * a)

    fm_ref[...] = jnp.zeros((H, W), jnp.float32)
    col = lax.broadcasted_iota(jnp.int32, (1, W), 1)

    def step(f, carry):
        y = rows_ref[0, f]
        left = lefts_ref[0, f]
        kw = widths_ref[0, f]
        ev = evs_ref[0, f]
        row = fm_ref[pl.ds(y, 1), :]
        off = (col - left) & (W - 1)
        covered = off < kw
        val = jnp.where((off == 0) | (off == kw - 1), ev, 1.0)
        new_row = jnp.where(covered, val, row)
        edge = (y == 0) | (y == H - 1)
        new_row = jnp.where(edge, jnp.ones_like(new_row), new_row)
        fm_ref[pl.ds(y, 1), :] = new_row
        return carry

    lax.fori_loop(0, F, step, 0, unroll=False)

    fm = fm_ref[...]
    sfm = jnp.sum(fm)
    sdot = jnp.sum(fm * a)

    mean = s1 / N
    var = (s2 - s1 * s1 / N) / (N - 1)
    std = jnp.sqrt(var)
    denom = std + jnp.where(std < EPS, EPS, 0.0)
    contrib = (sdot - mean * sfm) / (denom * (F * B))

    b = pl.program_id(0)

    @pl.when(b == 0)
    def _():
        out_ref[0, 0] = contrib

    @pl.when(b > 0)
    def _():
        out_ref[0, 0] += contrib


def kernel(y_pred, y_gt):
    lengths = jnp.asarray(_LEN_NP)
    evs_tab = jnp.asarray(_EV_NP)

    x_idx = jnp.rint(y_gt[:, :, 0] * (W - 1)).astype(jnp.int32)
    y_idx = jnp.rint(y_gt[:, :, 1] * (H - 1)).astype(jnp.int32)
    kw = lengths[y_idx]
    ev = evs_tab[y_idx]
    left = x_idx - kw // 2

    a = y_pred.reshape(B, H, W)

    out = pl.pallas_call(
        _nss_kernel,
        grid=(B,),
        in_specs=[
            pl.BlockSpec((1, F), lambda b: (b, 0), memory_space=pltpu.SMEM),
            pl.BlockSpec((1, F), lambda b: (b, 0), memory_space=pltpu.SMEM),
            pl.BlockSpec((1, F), lambda b: (b, 0), memory_space=pltpu.SMEM),
            pl.BlockSpec((1, F), lambda b: (b, 0), memory_space=pltpu.SMEM),
            pl.BlockSpec((1, H, W), lambda b: (b, 0, 0)),
        ],
        out_specs=pl.BlockSpec((1, 1), lambda b: (0, 0), memory_space=pltpu.SMEM),
        out_shape=jax.ShapeDtypeStruct((1, 1), jnp.float32),
        scratch_shapes=[pltpu.VMEM((H, W), jnp.float32)],
    )(y_idx, left, kw, ev, a)
    return out[0, 0]


# speedup vs baseline: 29.0800x; 29.0800x over previous
import math

import jax
import jax.numpy as jnp
import numpy as np
from jax import lax
from jax.experimental import pallas as pl
from jax.experimental.pallas import tpu as pltpu

H, W = 512, 1024
EPS = 1e-05
B, F = 64, 50
N = H * W


def _row_tables():
    thetas = np.linspace(0.5, H - 0.5, num=H) * math.pi / H
    weight = 1.0 / np.sin(thetas)
    residual = weight % 2
    mask = residual >= 1
    residual[mask] -= 1
    residual[~mask] += 1
    n_ones = (weight - residual).astype(np.int32)
    edge_values = ((weight - n_ones) / 2).astype(np.float32)
    lengths = n_ones + 2
    return lengths.astype(np.int32), edge_values


_LEN_NP, _EV_NP = _row_tables()


def _nss_kernel(rows_ref, lefts_ref, widths_ref, evs_ref, a_ref, out_ref, fm_ref):
    a = a_ref[0]
    s1 = jnp.sum(a)
    s2 = jnp.sum(a * a)

    fm_ref[...] = jnp.zeros((H, W), jnp.float32)
    col = lax.broadcasted_iota(jnp.int32, (1, W), 1)

    def step(f, carry):
        y = rows_ref[0, 0, f]
        left = lefts_ref[0, 0, f]
        kw = widths_ref[0, 0, f]
        ev = evs_ref[0, 0, f]
        row = fm_ref[pl.ds(y, 1), :]
        off = (col - left) & (W - 1)
        covered = off < kw
        val = jnp.where((off == 0) | (off == kw - 1), ev, 1.0)
        new_row = jnp.where(covered, val, row)
        edge = (y == 0) | (y == H - 1)
        new_row = jnp.where(edge, jnp.ones_like(new_row), new_row)
        fm_ref[pl.ds(y, 1), :] = new_row
        return carry

    lax.fori_loop(0, F, step, 0, unroll=False)

    fm = fm_ref[...]
    sfm = jnp.sum(fm)
    sdot = jnp.sum(fm * a)

    mean = s1 / N
    var = (s2 - s1 * s1 / N) / (N - 1)
    std = jnp.sqrt(var)
    denom = std + jnp.where(std < EPS, EPS, 0.0)
    contrib = (sdot - mean * sfm) / (denom * (F * B))

    b = pl.program_id(0)

    @pl.when(b == 0)
    def _():
        out_ref[0, 0] = contrib

    @pl.when(b > 0)
    def _():
        out_ref[0, 0] += contrib


def kernel(y_pred, y_gt):
    lengths = jnp.asarray(_LEN_NP)
    evs_tab = jnp.asarray(_EV_NP)

    x_idx = jnp.rint(y_gt[:, :, 0] * (W - 1)).astype(jnp.int32)
    y_idx = jnp.rint(y_gt[:, :, 1] * (H - 1)).astype(jnp.int32)
    kw = lengths[y_idx]
    ev = evs_tab[y_idx]
    left = x_idx - kw // 2

    a = y_pred.reshape(B, H, W)

    out = pl.pallas_call(
        _nss_kernel,
        grid=(B,),
        in_specs=[
            pl.BlockSpec((1, 1, F), lambda b: (b, 0, 0), memory_space=pltpu.SMEM),
            pl.BlockSpec((1, 1, F), lambda b: (b, 0, 0), memory_space=pltpu.SMEM),
            pl.BlockSpec((1, 1, F), lambda b: (b, 0, 0), memory_space=pltpu.SMEM),
            pl.BlockSpec((1, 1, F), lambda b: (b, 0, 0), memory_space=pltpu.SMEM),
            pl.BlockSpec((1, H, W), lambda b: (b, 0, 0)),
        ],
        out_specs=pl.BlockSpec((1, 1), lambda b: (0, 0), memory_space=pltpu.SMEM),
        out_shape=jax.ShapeDtypeStruct((1, 1), jnp.float32),
        scratch_shapes=[pltpu.VMEM((H, W), jnp.float32)],
    )(
        y_idx.reshape(B, 1, F),
        left.reshape(B, 1, F),
        kw.reshape(B, 1, F),
        ev.reshape(B, 1, F),
        a,
    )
    return out[0, 0]
